# baseline (speedup 1.0000x reference)
import jax
import jax.numpy as jnp
import numpy as np
from jax import lax
from jax.experimental import pallas as pl
from jax.experimental.pallas import tpu as pltpu
from jax.experimental.pallas import tpu_sc as plsc

_ROWS = 64
_COLS = 32768
_NCHUNKS = _COLS // 16
_NBINS = 4096
_MASK31 = np.int32(0x7FFFFFFF)


def _keys_of(x):
    xi = lax.bitcast_convert_type(x, jnp.int32)
    return xi ^ ((xi >> 31) & _MASK31)


def _sc_select_body(scores_hbm, k_hbm, out_hbm, row_v, cand_v, hist_v, kv_v,
                    thr_v):
    nc = jax.lax.axis_size("c")
    cid = lax.axis_index("c")
    sid = lax.axis_index("s")
    wid = sid * nc + cid

    pltpu.sync_copy(k_hbm, kv_v)
    kk = jnp.max(kv_v[...])
    iota = jnp.arange(16, dtype=jnp.int32)
    zeros16 = jnp.zeros((16,), jnp.int32)

    for rr in range(_ROWS // 32):
        r = wid * (_ROWS // 32) + rr
        pltpu.sync_copy(scores_hbm.at[r], row_v)

        def zero_step(i, _):
            hist_v[pl.ds(i * 16, 16)] = zeros16
            return 0

        lax.fori_loop(0, _NBINS // 16, zero_step, 0)

        def hist_step(i, _):
            key = _keys_of(row_v[pl.ds(i * 16, 16)])
            b = (key >> 20) + jnp.int32(2048)
            cnt, last = plsc.scan_count(b)
            plsc.addupdate_scatter(hist_v, [b], cnt, mask=last)
            return 0

        lax.fori_loop(0, _NCHUNKS, hist_step, 0)

        def scan_step(t, carry):
            total, b0, kin = carry
            j = jnp.int32(_NBINS // 16 - 1) - t
            v = hist_v[pl.ds(j * 16, 16)]
            rv = lax.rev(v, (0,))
            c = plsc.cumsum(rv) + total
            hit = jnp.where(c >= kk, jnp.int32(1), jnp.int32(0))
            idx = jnp.sum(jnp.where(plsc.cumsum(hit) == 0, jnp.int32(1),
                                    jnp.int32(0)))
            found = jnp.logical_and(b0 < 0, idx < 16)
            sel = iota == idx
            a = jnp.sum(jnp.where(sel, c, jnp.int32(0)))
            hb = jnp.sum(jnp.where(sel, rv, jnp.int32(0)))
            b0 = jnp.where(found, j * 16 + 15 - idx, b0)
            kin = jnp.where(found, kk - (a - hb), kin)
            return total + jnp.sum(v), b0, kin

        _, b0, kin = lax.fori_loop(
            0, _NBINS // 16, scan_step,
            (jnp.int32(0), jnp.int32(-1), jnp.int32(0)))

        def compact_step(i, off):
            key = _keys_of(row_v[pl.ds(i * 16, 16)])
            b = (key >> 20) + jnp.int32(2048)
            m = b == b0
            plsc.store_compressed(cand_v.at[pl.ds(off, 16)], key, mask=m)
            return off + jnp.sum(jnp.where(m, jnp.int32(1), jnp.int32(0)))

        m_cnt = lax.fori_loop(0, _NCHUNKS, compact_step, jnp.int32(0))
        nch = (m_cnt + 15) >> 4

        lo0 = (b0 - jnp.int32(2048)) << 20

        def bs_step(_, carry):
            lo, hi = carry
            x_and = lo & hi
            x_xor = lo ^ hi
            mid = x_and + (x_xor >> 1) + (x_xor & 1)

            def cnt_step(i, acc):
                v = cand_v[pl.ds(i * 16, 16)]
                ok = jnp.logical_and(i * 16 + iota < m_cnt, v >= mid)
                return acc + jnp.sum(jnp.where(ok, jnp.int32(1), jnp.int32(0)))

            cntv = lax.fori_loop(0, nch, cnt_step, jnp.int32(0))
            pred = cntv >= kin
            lo = jnp.where(pred, mid, lo)
            hi = jnp.where(pred, hi, mid - jnp.int32(1))
            return lo, hi

        tkey, _ = lax.fori_loop(0, 20, bs_step,
                                (lo0, lo0 + jnp.int32(0xFFFFF)))

        ti = tkey ^ ((tkey >> 31) & _MASK31)
        thr_v[...] = lax.bitcast_convert_type(
            jnp.broadcast_to(ti, (16,)), jnp.float32)
        pltpu.sync_copy(thr_v, out_hbm.at[r])


def _sc_select(scores, k_arr):
    mesh = plsc.VectorSubcoreMesh(core_axis_name="c", subcore_axis_name="s")
    f = pl.kernel(
        _sc_select_body,
        out_type=jax.ShapeDtypeStruct((_ROWS, 16), jnp.float32),
        mesh=mesh,
        compiler_params=pltpu.CompilerParams(needs_layout_passes=False),
        scratch_types=[
            pltpu.VMEM((_COLS,), jnp.float32),
            pltpu.VMEM((_COLS,), jnp.int32),
            pltpu.VMEM((_NBINS,), jnp.int32),
            pltpu.VMEM((16,), jnp.int32),
            pltpu.VMEM((16,), jnp.float32),
        ],
    )
    return f(scores, k_arr)


def _mask_body(t_ref, x_ref, o_ref):
    x = x_ref[...]
    t = t_ref[...][:, 0:1]
    o_ref[...] = (x >= t).astype(jnp.int32)


def _tc_mask(scores, thres_b):
    grid = (8,)
    return pl.pallas_call(
        _mask_body,
        grid=grid,
        in_specs=[
            pl.BlockSpec((_ROWS // 8, 128), lambda i: (i, 0)),
            pl.BlockSpec((_ROWS // 8, _COLS), lambda i: (i, 0)),
        ],
        out_specs=pl.BlockSpec((_ROWS // 8, _COLS), lambda i: (i, 0)),
        out_shape=jax.ShapeDtypeStruct((_ROWS, _COLS), jnp.int32),
    )(thres_b, scores)


def kernel(scores, k):
    k_arr = jnp.broadcast_to(jnp.asarray(k, jnp.int32), (16,))
    thres16 = _sc_select(scores, k_arr)
    thres_b = jnp.broadcast_to(thres16[:, :1], (_ROWS, 128))
    return _tc_mask(scores, thres_b)


# speedup vs baseline: 2.3265x; 2.3265x over previous
import jax
import jax.numpy as jnp
import numpy as np
from jax import lax
from jax.experimental import pallas as pl
from jax.experimental.pallas import tpu as pltpu
from jax.experimental.pallas import tpu_sc as plsc

_ROWS = 64
_COLS = 32768
_NCHUNKS = _COLS // 16
_SHIFT = 21
_NBINS = 1 << (32 - _SHIFT)
_LOWMASK = np.int32((1 << _SHIFT) - 1)
_MASK31 = np.int32(0x7FFFFFFF)
_U = 8


def _keys_of(x):
    xi = lax.bitcast_convert_type(x, jnp.int32)
    return xi ^ ((xi >> 31) & _MASK31)


def _bin_of(key):
    return (key >> _SHIFT) + jnp.int32(_NBINS // 2)


def _process_row(row_v, cand_v, hist_v, kk, iota, zeros16):
    def zero_step(i, _):
        base = i * 16 * _U
        for u in range(_U):
            hist_v[pl.ds(base + u * 16, 16)] = zeros16
        return 0

    lax.fori_loop(0, _U * _NBINS // (16 * _U), zero_step, 0)

    def hist_step(i, bmax_v):
        base = i * 16 * _U
        bs = []
        for u in range(_U):
            key = _keys_of(row_v[pl.ds(base + u * 16, 16)])
            bs.append(_bin_of(key))
        for u in range(_U):
            cnt, last = plsc.scan_count(bs[u])
            plsc.addupdate_scatter(hist_v, [bs[u] + jnp.int32(u * _NBINS)],
                                   cnt, mask=last)
        m01 = jnp.maximum(jnp.maximum(bs[0], bs[1]), jnp.maximum(bs[2], bs[3]))
        m45 = jnp.maximum(jnp.maximum(bs[4], bs[5]), jnp.maximum(bs[6], bs[7]))
        return jnp.maximum(bmax_v, jnp.maximum(m01, m45))

    bmax_v = lax.fori_loop(0, _NCHUNKS // _U, hist_step,
                           jnp.full((16,), -2**31, jnp.int32))
    binmax = jnp.max(bmax_v)

    jmax = binmax >> 4

    def scan_step(t, carry):
        total, b0, kin = carry
        j = jmax - t
        v = hist_v[pl.ds(j * 16, 16)]
        for u in range(1, _U):
            v = v + hist_v[pl.ds(u * _NBINS + j * 16, 16)]
        rv = lax.rev(v, (0,))
        c = plsc.cumsum(rv) + total
        hit = jnp.where(c >= kk, jnp.int32(1), jnp.int32(0))
        idx = jnp.sum(jnp.where(plsc.cumsum(hit) == 0, jnp.int32(1),
                                jnp.int32(0)))
        found = jnp.logical_and(b0 < 0, idx < 16)
        sel = iota == idx
        a = jnp.sum(jnp.where(sel, c, jnp.int32(0)))
        hb = jnp.sum(jnp.where(sel, rv, jnp.int32(0)))
        b0 = jnp.where(found, j * 16 + 15 - idx, b0)
        kin = jnp.where(found, kk - (a - hb), kin)
        return total + jnp.sum(v), b0, kin

    _, b0, kin = lax.fori_loop(
        0, jmax + 1, scan_step, (jnp.int32(0), jnp.int32(-1), jnp.int32(0)))

    def compact_step(i, off):
        base = i * 16 * _U
        ms, keys = [], []
        for u in range(_U):
            key = _keys_of(row_v[pl.ds(base + u * 16, 16)])
            keys.append(key)
            ms.append(_bin_of(key) == b0)
        ps = [jnp.sum(jnp.where(m, jnp.int32(1), jnp.int32(0))) for m in ms]
        offs = [off]
        for u in range(_U):
            offs.append(offs[u] + ps[u])
        for u in range(_U):
            plsc.store_compressed(cand_v.at[pl.ds(offs[u], 16)], keys[u],
                                  mask=ms[u])
        return offs[_U]

    m_cnt = lax.fori_loop(0, _NCHUNKS // _U, compact_step, jnp.int32(0))
    nch = (m_cnt + 15) >> 4

    lo0 = (b0 - jnp.int32(_NBINS // 2)) << _SHIFT

    def bs_step(_, carry):
        lo, hi = carry
        x_and = lo & hi
        x_xor = lo ^ hi
        mid = x_and + (x_xor >> 1) + (x_xor & 1)

        def cnt_step(i, acc):
            v = cand_v[pl.ds(i * 16, 16)]
            ok = jnp.logical_and(i * 16 + iota < m_cnt, v >= mid)
            return acc + jnp.sum(jnp.where(ok, jnp.int32(1), jnp.int32(0)))

        cntv = lax.fori_loop(0, nch, cnt_step, jnp.int32(0))
        pred = cntv >= kin
        lo = jnp.where(pred, mid, lo)
        hi = jnp.where(pred, hi, mid - jnp.int32(1))
        return lo, hi

    tkey, _ = lax.fori_loop(0, _SHIFT, bs_step, (lo0, lo0 + _LOWMASK))

    ti = tkey ^ ((tkey >> 31) & _MASK31)
    thres = lax.bitcast_convert_type(jnp.broadcast_to(ti, (16,)), jnp.float32)

    one_f = lax.bitcast_convert_type(jnp.full((16,), 1, jnp.int32),
                                     jnp.float32)
    zero_f = lax.bitcast_convert_type(zeros16, jnp.float32)

    def mask_step(i, _):
        base = i * 16 * _U
        for u in range(_U):
            x = row_v[pl.ds(base + u * 16, 16)]
            row_v[pl.ds(base + u * 16, 16)] = jnp.where(x >= thres, one_f,
                                                        zero_f)
        return 0

    lax.fori_loop(0, _NCHUNKS // _U, mask_step, 0)


def _sc_body(scores_hbm, k_hbm, out_hbm, row_a, row_b, cand_v, hist_v, kv_v,
             sem_a, sem_b):
    nc = jax.lax.axis_size("c")
    cid = lax.axis_index("c")
    sid = lax.axis_index("s")
    wid = sid * nc + cid
    r0 = wid * 2
    r1 = r0 + 1

    in_a = pltpu.async_copy(scores_hbm.at[r0], row_a, sem_a)
    in_b = pltpu.async_copy(scores_hbm.at[r1], row_b, sem_b)
    pltpu.sync_copy(k_hbm, kv_v)
    kk = jnp.max(kv_v[...])
    iota = jnp.arange(16, dtype=jnp.int32)
    zeros16 = jnp.zeros((16,), jnp.int32)

    in_a.wait()
    _process_row(row_a, cand_v, hist_v, kk, iota, zeros16)
    out_a = pltpu.async_copy(row_a, out_hbm.at[r0], sem_a)
    in_b.wait()
    _process_row(row_b, cand_v, hist_v, kk, iota, zeros16)
    out_b = pltpu.async_copy(row_b, out_hbm.at[r1], sem_b)
    out_a.wait()
    out_b.wait()


def kernel(scores, k):
    k_arr = jnp.broadcast_to(jnp.asarray(k, jnp.int32), (16,))
    mesh = plsc.VectorSubcoreMesh(core_axis_name="c", subcore_axis_name="s")
    f = pl.kernel(
        _sc_body,
        out_type=jax.ShapeDtypeStruct((_ROWS, _COLS), jnp.float32),
        mesh=mesh,
        compiler_params=pltpu.CompilerParams(needs_layout_passes=False),
        scratch_types=[
            pltpu.VMEM((_COLS,), jnp.float32),
            pltpu.VMEM((_COLS,), jnp.float32),
            pltpu.VMEM((_COLS,), jnp.int32),
            pltpu.VMEM((_U * _NBINS,), jnp.int32),
            pltpu.VMEM((16,), jnp.int32),
            pltpu.SemaphoreType.DMA,
            pltpu.SemaphoreType.DMA,
        ],
    )
    out_f = f(scores, k_arr)
    return lax.bitcast_convert_type(out_f, jnp.int32)
